# baseline (speedup 1.0000x reference)
import functools

import jax
import jax.numpy as jnp
from jax import lax
from jax.experimental import pallas as pl
from jax.experimental.pallas import tpu as pltpu
from jax.experimental.pallas import tpu_sc as plsc

NC = 2
NS = 16
L = 16
NW = NC * NS

B = 16384
D = 32
BPW = B // NW
NG = 7813
GPW = 245
CAP = 1008
SCAP = 256 * L
KQ = 14
RS = 32


def _gather_kernel(uidx_hbm, iidx_hbm, u_t_hbm, i_t_hbm,
                   urows_hbm, vrows_hbm,
                   idx_vm, idx2_vm, sel_r, sel_b, sel2_r, sel2_b,
                   slots_r, slots_b, slots2_r, slots2_b,
                   olist_sm, olist2_sm, hist_sm, hist2_sm, gbuf, rowstage,
                   gsem, wsem):
    wid = lax.axis_index("s") * NC + lax.axis_index("c")
    rows0 = lax.iota(jnp.int32, L)
    SI = B // L // 8

    pltpu.sync_copy(uidx_hbm, idx_vm)
    pltpu.sync_copy(iidx_hbm, idx2_vm)

    def zero_hists(t, _):
        hist_sm[t] = 0
        hist2_sm[t] = 0
        return 0
    lax.fori_loop(0, 256, zero_hists, 0)

    def sel_iter(q, cnt, src_vm, dst_r, dst_b):
        chunks, csums = [], []
        for t in range(8):
            chunk = src_vm[pl.ds((q * 8 + t) * L, L)]
            g = lax.shift_right_logical(chunk, 7)
            own = (g & 31) == wid
            csums.append(plsc.cumsum(jnp.where(own, 1, 0).astype(jnp.int32)))
            chunks.append((chunk, own))
        for t in range(8):
            chunk, own = chunks[t]
            bpos = (q * 8 + t) * L + rows0
            dest = cnt + csums[t] - 1
            okm = own & (dest < CAP)
            plsc.store_scatter(dst_r, [dest], chunk, mask=okm)
            plsc.store_scatter(dst_b, [dest], bpos, mask=okm)
            cnt = cnt + csums[t][L - 1]
        return cnt

    def bin_iter(k, cnt, src_r, src_b, dst_r, dst_b, hist):
        chunk = src_r[pl.ds(k * L, L)]
        bchunk = src_b[pl.ds(k * L, L)]
        valid = (k * L + rows0) < cnt
        lgv = jnp.where(valid, lax.shift_right_logical(chunk, 12), 255)
        posv = jnp.zeros((L,), jnp.int32)
        for j in range(L):
            lg = lgv[j]
            c = hist[lg]
            hist[lg] = c + 1
            pj = jnp.full((L,), lg * L + c, jnp.int32)
            posv = jnp.where(rows0 == j, pj, posv)
        plsc.store_scatter(dst_r, [posv], chunk, mask=valid)
        plsc.store_scatter(dst_b, [posv], bchunk, mask=valid)

    def olist_build(hist, olist):
        def ol_body(t, n):
            c = hist[t]

            def put():
                olist[n] = t
                return n + 1
            return jax.lax.cond(c > 0, put, lambda: n)
        return lax.fori_loop(0, GPW, ol_body, jnp.int32(0))

    def fire(oi, t_hbm, olist):
        gi = olist[oi]
        gcol = (gi * 32 + wid) * 128
        gcol = pl.multiple_of(gcol, 128)
        pltpu.async_copy(t_hbm.at[:, pl.ds(gcol, 128)],
                         gbuf.at[lax.rem(oi, KQ)], gsem)

    def group_work(oi, gwc, t_hbm, rows_out_hbm, olist, hist, ocnt):
        slot = lax.rem(oi, KQ)
        pltpu.make_async_copy(t_hbm.at[:, pl.ds(0, 128)],
                              gbuf.at[slot], gsem).wait()
        gi = olist[oi]
        cg = hist[gi]
        sbase = gi * L

        @pl.when(oi + KQ < ocnt)
        def _():
            fire(oi + KQ, t_hbm, olist)

        def member(m, _):
            w = gwc + m

            @pl.when(w >= RS)
            def _():
                pltpu.make_async_copy(rowstage.at[0],
                                      rows_out_hbm.at[pl.ds(0, D)],
                                      wsem).wait()
            rmv = plsc.load_gather(slots_r if rows_out_hbm is urows_hbm
                                   else slots2_r,
                                   [jnp.full((L,), sbase, jnp.int32) + m])
            bmv = plsc.load_gather(slots_b if rows_out_hbm is urows_hbm
                                   else slots2_b,
                                   [jnp.full((L,), sbase, jnp.int32) + m])
            col = rmv & 127
            sv = jnp.full((L,), slot, jnp.int32)
            u0 = plsc.load_gather(gbuf, [sv, rows0, col])
            u1 = plsc.load_gather(gbuf, [sv, rows0 + L, col])
            rs = lax.rem(w, RS)
            rowstage[rs, pl.ds(0, L)] = u0
            rowstage[rs, pl.ds(L, L)] = u1
            b0 = bmv[0] * D
            pltpu.async_copy(rowstage.at[rs],
                             rows_out_hbm.at[pl.ds(b0, D)], wsem)
            return 0
        lax.fori_loop(0, cg, member, 0)
        return gwc + cg

    def final_drain(rows_out_hbm, wcnt):
        def fd(m, _):
            pltpu.make_async_copy(rowstage.at[0],
                                  rows_out_hbm.at[pl.ds(0, D)], wsem).wait()
            return 0
        lax.fori_loop(0, jnp.minimum(wcnt, RS), fd, 0)

    def sel_u(q, cnt):
        return sel_iter(q, cnt, idx_vm, sel_r, sel_b)
    cnt_u = lax.fori_loop(0, SI, sel_u, jnp.int32(0))

    def bin_u(k, _):
        bin_iter(k, cnt_u, sel_r, sel_b, slots_r, slots_b, hist_sm)
        return 0
    lax.fori_loop(0, (cnt_u + L - 1) // L, bin_u, 0)
    ocnt_u = olist_build(hist_sm, olist_sm)

    def prime_u(oi, _):
        @pl.when(oi < ocnt_u)
        def _():
            fire(oi, u_t_hbm, olist_sm)
        return 0
    lax.fori_loop(0, KQ, prime_u, 0)

    NI = SI + 64

    def walk_u(oi, carry):
        gwc, cnt_i = carry
        gwc = jax.lax.cond(
            oi < ocnt_u,
            lambda: group_work(oi, gwc, u_t_hbm, urows_hbm,
                               olist_sm, hist_sm, ocnt_u),
            lambda: gwc)
        cnt_i = jax.lax.cond(
            oi < SI,
            lambda: sel_iter(oi, cnt_i, idx2_vm, sel2_r, sel2_b),
            lambda: cnt_i)

        @pl.when((oi >= SI) & (oi - SI < (cnt_i + L - 1) // L))
        def _():
            bin_iter(oi - SI, cnt_i, sel2_r, sel2_b,
                     slots2_r, slots2_b, hist2_sm)
        return (gwc, cnt_i)

    loop_n = jnp.maximum(ocnt_u, NI)
    wcnt_u, _cnt_i = lax.fori_loop(0, loop_n, walk_u,
                                   (jnp.int32(0), jnp.int32(0)))
    final_drain(urows_hbm, wcnt_u)

    ocnt_i = olist_build(hist2_sm, olist2_sm)

    def prime_i(oi, _):
        @pl.when(oi < ocnt_i)
        def _():
            fire(oi, i_t_hbm, olist2_sm)
        return 0
    lax.fori_loop(0, KQ, prime_i, 0)

    def walk_i(oi, gwc):
        return group_work(oi, gwc, i_t_hbm, vrows_hbm,
                          olist2_sm, hist2_sm, ocnt_i)
    wcnt_i = lax.fori_loop(0, ocnt_i, walk_i, jnp.int32(0))
    final_drain(vrows_hbm, wcnt_i)


def _dot_kernel(urows_hbm, vrows_hbm, out_hbm, u_vm, v_vm, out_v, sem1, sem2):
    wid = lax.axis_index("s") * NC + lax.axis_index("c")
    base = wid * BPW
    c1 = pltpu.async_copy(urows_hbm.at[pl.ds(base * D, BPW * D)], u_vm, sem1)
    c2 = pltpu.async_copy(vrows_hbm.at[pl.ds(base * D, BPW * D)], v_vm, sem2)
    c1.wait()
    c2.wait()
    rows0 = lax.iota(jnp.int32, L)

    def gbody(g, _):
        fbase = g * L * D
        acc = jnp.zeros((L,), jnp.float32)
        for d in range(D):
            fidx = fbase + rows0 * D + d
            u = plsc.load_gather(u_vm, [fidx])
            v = plsc.load_gather(v_vm, [fidx])
            acc = acc + u * v
        out_v[pl.ds(g * L, L)] = acc
        return 0
    lax.fori_loop(0, BPW // L, gbody, 0)
    pltpu.sync_copy(out_v, out_hbm.at[pl.ds(base, BPW)])


@jax.jit
def _run(user_idx, item_idx, user_w, item_w):
    mesh = plsc.VectorSubcoreMesh(core_axis_name="c", subcore_axis_name="s")
    k1 = functools.partial(
        pl.kernel,
        out_type=(jax.ShapeDtypeStruct((B * D,), jnp.float32),
                  jax.ShapeDtypeStruct((B * D,), jnp.float32)),
        mesh=mesh,
        compiler_params=pltpu.CompilerParams(
            needs_layout_passes=False, use_tc_tiling_on_sc=True),
        scratch_types=[
            pltpu.VMEM((B,), jnp.int32),
            pltpu.VMEM((B,), jnp.int32),
            pltpu.VMEM((CAP,), jnp.int32),
            pltpu.VMEM((CAP,), jnp.int32),
            pltpu.VMEM((CAP,), jnp.int32),
            pltpu.VMEM((CAP,), jnp.int32),
            pltpu.VMEM((SCAP,), jnp.int32),
            pltpu.VMEM((SCAP,), jnp.int32),
            pltpu.VMEM((SCAP,), jnp.int32),
            pltpu.VMEM((SCAP,), jnp.int32),
            pltpu.SMEM((256,), jnp.int32),
            pltpu.SMEM((256,), jnp.int32),
            pltpu.SMEM((256,), jnp.int32),
            pltpu.SMEM((256,), jnp.int32),
            pltpu.VMEM((KQ, D, 128), jnp.float32),
            pltpu.VMEM((RS, D), jnp.float32),
            pltpu.SemaphoreType.DMA,
            pltpu.SemaphoreType.DMA,
        ],
    )(_gather_kernel)
    urows, vrows = k1(user_idx, item_idx, user_w.T, item_w.T)

    k2 = functools.partial(
        pl.kernel,
        out_type=jax.ShapeDtypeStruct((B,), jnp.float32),
        mesh=mesh,
        compiler_params=pltpu.CompilerParams(
            needs_layout_passes=False, use_tc_tiling_on_sc=True),
        scratch_types=[
            pltpu.VMEM((BPW * D,), jnp.float32),
            pltpu.VMEM((BPW * D,), jnp.float32),
            pltpu.VMEM((BPW,), jnp.float32),
            pltpu.SemaphoreType.DMA,
            pltpu.SemaphoreType.DMA,
        ],
    )(_dot_kernel)
    return k2(urows, vrows)


def kernel(user_idx, item_idx, user_w, item_w):
    return _run(user_idx, item_idx, user_w, item_w)
